# Initial kernel scaffold; baseline (speedup 1.0000x reference)
import jax
import jax.numpy as jnp
from jax.experimental import pallas as pl


def kernel(prob, _k_head):
    raise NotImplementedError("write your pallas kernel here")


# speedup vs baseline: 1.3740x; 1.3740x over previous
import jax
import jax.numpy as jnp
from jax.experimental import pallas as pl
from jax.experimental.pallas import tpu as pltpu

BATCH = 4096
K = 1000
ACTION_SIZE = 128
KP = 1024
NCH = KP // 128
RB = 512


def _tanh_body(kh_ref, th_ref):
    th_ref[...] = jnp.tanh(kh_ref[...])


def _sample_body(pv_ref, u_ref, th_ref, action_ref, sp_ref, L_ref):
    def step(i, loc):
        loc = loc + pv_ref[pl.ds(i, 1)]
        L_ref[pl.ds(i, 1)] = loc
        return loc

    jax.lax.fori_loop(0, 128, step, jnp.zeros((1, NCH, RB), jnp.float32))

    L = L_ref[...]
    ltot = L[127:128]
    run = jnp.zeros((1, 1, RB), jnp.float32)
    parts = [run]
    for c in range(NCH - 1):
        run = run + ltot[:, c:c + 1, :]
        parts.append(run)
    carr = jnp.concatenate(parts, axis=1)
    total = run + ltot[:, NCH - 1:NCH, :]

    thresh = u_ref[...].reshape(1, 1, RB) * total
    y = L + carr
    pos_i = jax.lax.broadcasted_iota(jnp.int32, (128, NCH, 1), 0)
    pos_c = jax.lax.broadcasted_iota(jnp.int32, (128, NCH, 1), 1)
    g = pos_i + 128 * pos_c
    valid = g < K
    cnt = jnp.sum(((y < thresh) & valid).astype(jnp.int32), axis=(0, 1),
                  keepdims=True)
    ind = jnp.minimum(cnt, K - 1)

    oh = (g == ind).astype(jnp.float32)
    sp_ref[...] = jnp.sum(oh * pv_ref[...], axis=(0, 1)).reshape(1, RB)
    action_ref[...] = jax.lax.dot_general(
        oh.reshape(KP, RB), th_ref[...],
        dimension_numbers=(((0,), (0,)), ((), ())),
        preferred_element_type=jnp.float32,
        precision=jax.lax.Precision.HIGHEST)


def kernel(prob, _k_head):
    u = jax.random.uniform(jax.random.key(42), (BATCH, 1), dtype=jnp.float32)
    u2 = u.reshape(1, BATCH)

    pv = jnp.pad(prob, ((0, 0), (0, KP - K))) \
            .reshape(BATCH, NCH, 128).transpose(2, 1, 0)
    khp = jnp.pad(_k_head, ((0, KP - K), (0, 0))) \
            .reshape(NCH, 128, ACTION_SIZE).transpose(1, 0, 2) \
            .reshape(KP, ACTION_SIZE)

    th = pl.pallas_call(
        _tanh_body,
        out_shape=jax.ShapeDtypeStruct((KP, ACTION_SIZE), jnp.float32),
    )(khp)

    grid = (BATCH // RB,)
    action, sp = pl.pallas_call(
        _sample_body,
        grid=grid,
        in_specs=[
            pl.BlockSpec((128, NCH, RB), lambda i: (0, 0, i)),
            pl.BlockSpec((1, RB), lambda i: (0, i)),
            pl.BlockSpec((KP, ACTION_SIZE), lambda i: (0, 0)),
        ],
        out_specs=[
            pl.BlockSpec((RB, ACTION_SIZE), lambda i: (i, 0)),
            pl.BlockSpec((1, RB), lambda i: (0, i)),
        ],
        out_shape=[
            jax.ShapeDtypeStruct((BATCH, ACTION_SIZE), jnp.float32),
            jax.ShapeDtypeStruct((1, BATCH), jnp.float32),
        ],
        scratch_shapes=[pltpu.VMEM((128, NCH, RB), jnp.float32)],
    )(pv, u2, th)
    return (action, sp.reshape(BATCH, 1))


# speedup vs baseline: 1.4683x; 1.0686x over previous
import jax
import jax.numpy as jnp
from jax.experimental import pallas as pl
from jax.experimental.pallas import tpu as pltpu

BATCH = 4096
K = 1000
ACTION_SIZE = 128
KP = 1024
NCH = KP // 128
RB = 512


def _sample_body(pv_ref, u_ref, khp_ref, action_ref, sp_ref, L_ref):
    loc = pv_ref[0:1]
    L_ref[0:1] = loc
    for i in range(1, 128):
        loc = loc + pv_ref[i:i + 1]
        L_ref[i:i + 1] = loc

    L = L_ref[...]
    ltot = loc
    run = jnp.zeros((1, 1, RB), jnp.float32)
    parts = [run]
    for c in range(NCH - 1):
        run = run + ltot[:, c:c + 1, :]
        parts.append(run)
    carr = jnp.concatenate(parts, axis=1)
    total = run + ltot[:, NCH - 1:NCH, :]

    thresh = u_ref[...].reshape(1, 1, RB) * total
    y = L + carr
    pos_i = jax.lax.broadcasted_iota(jnp.int32, (128, NCH, 1), 0)
    pos_c = jax.lax.broadcasted_iota(jnp.int32, (128, NCH, 1), 1)
    g = pos_i + 128 * pos_c
    valid = g < K
    cnt = jnp.sum(((y < thresh) & valid).astype(jnp.int32), axis=(0, 1),
                  keepdims=True)
    ind = jnp.minimum(cnt, K - 1)

    oh = (g == ind).astype(jnp.float32)
    sp_ref[...] = jnp.sum(oh * pv_ref[...], axis=(0, 1)).reshape(1, RB)
    action_ref[...] = jax.lax.dot_general(
        oh.reshape(KP, RB), jnp.tanh(khp_ref[...]),
        dimension_numbers=(((0,), (0,)), ((), ())),
        preferred_element_type=jnp.float32,
        precision=jax.lax.Precision.HIGHEST)


def kernel(prob, _k_head):
    u = jax.random.uniform(jax.random.key(42), (BATCH, 1), dtype=jnp.float32)
    u2 = u.reshape(1, BATCH)

    pv = jnp.pad(prob, ((0, 0), (0, KP - K))) \
            .reshape(BATCH, NCH, 128).transpose(2, 1, 0)
    khp = jnp.pad(_k_head, ((0, KP - K), (0, 0))) \
            .reshape(NCH, 128, ACTION_SIZE).transpose(1, 0, 2) \
            .reshape(KP, ACTION_SIZE)

    grid = (BATCH // RB,)
    action, sp = pl.pallas_call(
        _sample_body,
        grid=grid,
        in_specs=[
            pl.BlockSpec((128, NCH, RB), lambda i: (0, 0, i)),
            pl.BlockSpec((1, RB), lambda i: (0, i)),
            pl.BlockSpec((KP, ACTION_SIZE), lambda i: (0, 0)),
        ],
        out_specs=[
            pl.BlockSpec((RB, ACTION_SIZE), lambda i: (i, 0)),
            pl.BlockSpec((1, RB), lambda i: (0, i)),
        ],
        out_shape=[
            jax.ShapeDtypeStruct((BATCH, ACTION_SIZE), jnp.float32),
            jax.ShapeDtypeStruct((1, BATCH), jnp.float32),
        ],
        scratch_shapes=[pltpu.VMEM((128, NCH, RB), jnp.float32)],
    )(pv, u2, khp)
    return (action, sp.reshape(BATCH, 1))


# speedup vs baseline: 1.8377x; 1.2516x over previous
import jax
import jax.numpy as jnp
from jax.experimental import pallas as pl
from jax.experimental.pallas import tpu as pltpu

BATCH = 4096
K = 1000
ACTION_SIZE = 128
KP = 1024
NCH = KP // 128
RB = 512


def _sample_body(p_ref, u_ref, khp_ref, action_ref, sp_ref, S_ref, L_ref):
    p = p_ref[...]
    for c in range(NCH):
        hi = min((c + 1) * 128, K)
        blkc = p[:, c * 128:hi]
        if hi - c * 128 < 128:
            blkc = jnp.concatenate(
                [blkc, jnp.zeros((RB, 128 - (hi - c * 128)), jnp.float32)],
                axis=1)
        S_ref[:, c, :] = blkc.T

    loc = S_ref[0:1]
    L_ref[0:1] = loc
    for i in range(1, 128):
        loc = loc + S_ref[i:i + 1]
        L_ref[i:i + 1] = loc

    L = L_ref[...]
    ltot = loc
    run = jnp.zeros((1, 1, RB), jnp.float32)
    parts = [run]
    for c in range(NCH - 1):
        run = run + ltot[:, c:c + 1, :]
        parts.append(run)
    carr = jnp.concatenate(parts, axis=1)
    total = run + ltot[:, NCH - 1:NCH, :]

    thresh = u_ref[...].reshape(1, 1, RB) * total
    y = L + carr
    pos_i = jax.lax.broadcasted_iota(jnp.int32, (128, NCH, 1), 0)
    pos_c = jax.lax.broadcasted_iota(jnp.int32, (128, NCH, 1), 1)
    g = pos_i + 128 * pos_c
    valid = g < K
    cnt = jnp.sum(((y < thresh) & valid).astype(jnp.int32), axis=(0, 1),
                  keepdims=True)
    ind = jnp.minimum(cnt, K - 1)

    oh = (g == ind).astype(jnp.float32)
    sp_ref[...] = jnp.sum(oh * S_ref[...], axis=(0, 1)).reshape(1, RB)
    action_ref[...] = jax.lax.dot_general(
        oh.reshape(KP, RB), jnp.tanh(khp_ref[...]),
        dimension_numbers=(((0,), (0,)), ((), ())),
        preferred_element_type=jnp.float32,
        precision=jax.lax.Precision.HIGHEST)


def kernel(prob, _k_head):
    u = jax.random.uniform(jax.random.key(42), (BATCH, 1), dtype=jnp.float32)
    u2 = u.reshape(1, BATCH)

    khp = jnp.pad(_k_head, ((0, KP - K), (0, 0))) \
            .reshape(NCH, 128, ACTION_SIZE).transpose(1, 0, 2) \
            .reshape(KP, ACTION_SIZE)

    grid = (BATCH // RB,)
    action, sp = pl.pallas_call(
        _sample_body,
        grid=grid,
        in_specs=[
            pl.BlockSpec((RB, K), lambda i: (i, 0)),
            pl.BlockSpec((1, RB), lambda i: (0, i)),
            pl.BlockSpec((KP, ACTION_SIZE), lambda i: (0, 0)),
        ],
        out_specs=[
            pl.BlockSpec((RB, ACTION_SIZE), lambda i: (i, 0)),
            pl.BlockSpec((1, RB), lambda i: (0, i)),
        ],
        out_shape=[
            jax.ShapeDtypeStruct((BATCH, ACTION_SIZE), jnp.float32),
            jax.ShapeDtypeStruct((1, BATCH), jnp.float32),
        ],
        scratch_shapes=[pltpu.VMEM((128, NCH, RB), jnp.float32),
                        pltpu.VMEM((128, NCH, RB), jnp.float32)],
    )(prob, u2, khp)
    return (action, sp.reshape(BATCH, 1))


# speedup vs baseline: 2.5318x; 1.3777x over previous
import jax
import jax.numpy as jnp
import numpy as np
from jax.experimental import pallas as pl
from jax.experimental.pallas import tpu as pltpu

BATCH = 4096
K = 1000
ACTION_SIZE = 128
KP = 1024
NCH = KP // 128
RB = 1024

_U2 = np.asarray(
    jax.random.uniform(jax.random.key(42), (BATCH, 1), dtype=jnp.float32)
).reshape(1, BATCH)


def _sample_body(p_ref, u_ref, kh_ref, action_ref, sp_ref, S_ref, L_ref):
    p = p_ref[...]
    for c in range(NCH):
        hi = min((c + 1) * 128, K)
        blkc = p[:, c * 128:hi]
        if hi - c * 128 < 128:
            blkc = jnp.concatenate(
                [blkc, jnp.zeros((RB, 128 - (hi - c * 128)), jnp.float32)],
                axis=1)
        S_ref[:, c, :] = blkc.T

    loc = S_ref[0:1]
    L_ref[0:1] = loc
    for i in range(1, 128):
        loc = loc + S_ref[i:i + 1]
        L_ref[i:i + 1] = loc

    L = L_ref[...]
    ltot = loc
    run = jnp.zeros((1, 1, RB), jnp.float32)
    parts = [run]
    for c in range(NCH - 1):
        run = run + ltot[:, c:c + 1, :]
        parts.append(run)
    carr = jnp.concatenate(parts, axis=1)
    total = run + ltot[:, NCH - 1:NCH, :]

    thresh = u_ref[...].reshape(1, 1, RB) * total
    y = L + carr
    pos_i = jax.lax.broadcasted_iota(jnp.int32, (128, NCH, 1), 0)
    pos_c = jax.lax.broadcasted_iota(jnp.int32, (128, NCH, 1), 1)
    g = pos_i + 128 * pos_c
    valid = g < K
    cnt = jnp.sum(((y < thresh) & valid).astype(jnp.int32), axis=(0, 1),
                  keepdims=True)
    ind = jnp.minimum(cnt, K - 1)

    oh = (g == ind).astype(jnp.float32)
    sp_ref[...] = jnp.sum(oh * S_ref[...], axis=(0, 1)).reshape(1, RB)

    pos_c2 = jax.lax.broadcasted_iota(jnp.int32, (NCH, 128, 1), 0)
    pos_i2 = jax.lax.broadcasted_iota(jnp.int32, (NCH, 128, 1), 1)
    g2 = 128 * pos_c2 + pos_i2
    oh2 = (g2 == ind.reshape(1, 1, RB)).astype(jnp.float32)
    th = jnp.concatenate(
        [jnp.tanh(kh_ref[...]),
         jnp.zeros((KP - K, ACTION_SIZE), jnp.float32)], axis=0)
    action_ref[...] = jax.lax.dot_general(
        oh2.reshape(KP, RB), th,
        dimension_numbers=(((0,), (0,)), ((), ())),
        preferred_element_type=jnp.float32)


def kernel(prob, _k_head):
    u2 = jnp.asarray(_U2)

    grid = (BATCH // RB,)
    action, sp = pl.pallas_call(
        _sample_body,
        grid=grid,
        in_specs=[
            pl.BlockSpec((RB, K), lambda i: (i, 0)),
            pl.BlockSpec((1, RB), lambda i: (0, i)),
            pl.BlockSpec((K, ACTION_SIZE), lambda i: (0, 0)),
        ],
        out_specs=[
            pl.BlockSpec((RB, ACTION_SIZE), lambda i: (i, 0)),
            pl.BlockSpec((1, RB), lambda i: (0, i)),
        ],
        out_shape=[
            jax.ShapeDtypeStruct((BATCH, ACTION_SIZE), jnp.float32),
            jax.ShapeDtypeStruct((1, BATCH), jnp.float32),
        ],
        scratch_shapes=[pltpu.VMEM((128, NCH, RB), jnp.float32),
                        pltpu.VMEM((128, NCH, RB), jnp.float32)],
    )(prob, u2, _k_head)
    return (action, sp.reshape(BATCH, 1))
